# Initial kernel scaffold; baseline (speedup 1.0000x reference)
import jax
import jax.numpy as jnp
from jax.experimental import pallas as pl


def kernel(r, R_nb_en, idx_en, en_scales, en_kernel, en_bias, W_beta, W_gamma_init, W_gamma_out, W_edge, b_edge, z_n):
    raise NotImplementedError("write your pallas kernel here")


# speedup vs baseline: 6.0763x; 6.0763x over previous
import jax
import jax.numpy as jnp
from jax.experimental import pallas as pl

N_NUC = 64
N_ELEC = 4096
NB = 16
CUTOFF = 5.0
F0 = 32
F1 = 16
FEATURE_DIM = 64
N_ENV = 8
N_FEAT = 4

P = N_ELEC * NB
BP = 2048

_C_K0 = 0
_C_BIAS = 4 * F0
_C_SCALE = _C_BIAS + F0
_C_Z = _C_SCALE + N_ENV
_C_END = _C_Z + FEATURE_DIM


def _block_kernel(rp_ref, Rf_ref, idx_ref, T_ref, Wb_ref, Wg_ref, We_ref,
                  gi_ref, go_ref, ed_ref):
    rp = rp_ref[...]
    Rf = Rf_ref[...]
    idx = idx_ref[...]

    lanes = jax.lax.broadcasted_iota(jnp.int32, (BP, N_NUC), 1)
    oh = (idx == lanes).astype(jnp.float32)
    G = jnp.dot(oh, T_ref[...], preferred_element_type=jnp.float32)

    diff = rp - Rf
    dist2 = jnp.sum(diff * diff, axis=1, keepdims=True)
    dist = jnp.sqrt(dist2)
    feat = jnp.concatenate([dist, diff], axis=1)

    pre_h = G[:, _C_BIAS:_C_BIAS + F0]
    for f in range(N_FEAT):
        pre_h = pre_h + feat[:, f:f + 1] * G[:, f * F0:(f + 1) * F0]
    h = jnp.tanh(pre_h)

    scales = G[:, _C_SCALE:_C_SCALE + N_ENV]
    env = jnp.exp(-jnp.square(dist / scales))

    x = dist * (1.0 / CUTOFF)
    window = jnp.where(x < 1.0, jnp.square(1.0 - x) * (1.0 + 2.0 * x), 0.0)

    hb = jnp.concatenate([h, env], axis=1)
    beta = jnp.dot(hb, Wb_ref[...], preferred_element_type=jnp.float32)
    beta = beta * window

    gam = jnp.dot(beta, Wg_ref[...], preferred_element_type=jnp.float32)
    gi_ref[...] = gam[:, :FEATURE_DIM]
    go_ref[...] = gam[:, FEATURE_DIM:]

    s = jnp.log1p(dist) / dist
    inp = feat * s
    ones = jnp.ones((BP, 1), dtype=jnp.float32)
    zeros3 = jnp.zeros((BP, 3), dtype=jnp.float32)
    inp8 = jnp.concatenate([inp, ones, zeros3], axis=1)
    edge = jnp.dot(inp8, We_ref[...], preferred_element_type=jnp.float32)
    ed_ref[...] = edge + G[:, _C_Z:_C_END]


def kernel(r, R_nb_en, idx_en, en_scales, en_kernel, en_bias, W_beta,
           W_gamma_init, W_gamma_out, W_edge, b_edge, z_n):
    rp = jnp.broadcast_to(r[:, None, :], (N_ELEC, NB, 3)).reshape(P, 3)
    Rf = R_nb_en.reshape(P, 3)
    idxc = idx_en.reshape(P, 1)

    T = jnp.concatenate(
        [en_kernel.reshape(N_NUC, N_FEAT * F0), en_bias, en_scales, z_n],
        axis=1)
    Wg = jnp.concatenate([W_gamma_init, W_gamma_out], axis=1)
    We = jnp.concatenate(
        [W_edge, b_edge[None, :], jnp.zeros((3, FEATURE_DIM), jnp.float32)],
        axis=0)

    grid = (P // BP,)
    out_shape = [
        jax.ShapeDtypeStruct((P, FEATURE_DIM), jnp.float32),
        jax.ShapeDtypeStruct((P, FEATURE_DIM), jnp.float32),
        jax.ShapeDtypeStruct((P, FEATURE_DIM), jnp.float32),
    ]
    row_spec = lambda w: pl.BlockSpec((BP, w), lambda i: (i, 0))
    full_spec = lambda a, b: pl.BlockSpec((a, b), lambda i: (0, 0))
    gi, go, ed = pl.pallas_call(
        _block_kernel,
        grid=grid,
        in_specs=[
            row_spec(3), row_spec(3), row_spec(1),
            full_spec(N_NUC, _C_END),
            full_spec(F0 + N_ENV, F1),
            full_spec(F1, 2 * FEATURE_DIM),
            full_spec(8, FEATURE_DIM),
        ],
        out_specs=[row_spec(FEATURE_DIM)] * 3,
        out_shape=out_shape,
    )(rp, Rf, idxc, T, W_beta, Wg, We)
    shp = (N_ELEC, NB, FEATURE_DIM)
    return (gi.reshape(shp), go.reshape(shp), ed.reshape(shp))
